# baseline (speedup 1.0000x reference)
import functools

import jax
import jax.numpy as jnp
from jax import lax
from jax.experimental import pallas as pl
from jax.experimental.pallas import tpu as pltpu
from jax.experimental.pallas import tpu_sc as plsc

_F = 26
_V1 = 100001
_B = 16384
_NC, _NS, _L = 2, 16, 16
_NW = _NC * _NS
_BPW = _B // _NW
_CH = _BPW // _L

_mesh = plsc.VectorSubcoreMesh(core_axis_name="c", subcore_axis_name="s")


@functools.partial(
    pl.kernel,
    out_type=jax.ShapeDtypeStruct((_B,), jnp.float32),
    mesh=_mesh,
    scratch_types=[
        pltpu.VMEM((_F * _BPW,), jnp.int32),
        pltpu.VMEM((_F * _BPW,), jnp.float32),
        pltpu.VMEM((_BPW,), jnp.float32),
        pltpu.SemaphoreType.DMA,
        pltpu.SemaphoreType.DMA,
    ],
    compiler_params=pltpu.CompilerParams(use_tc_tiling_on_sc=False),
)
def _lookup_sum(idx_hbm, *refs):
    tabs = refs[:_F]
    out_hbm, idx_v, gath_v, out_v, sem, sem2 = refs[_F:]
    wid = lax.axis_index("s") * _NC + lax.axis_index("c")
    base = wid * _BPW

    stage = [
        pltpu.async_copy(
            idx_hbm.at[f, pl.ds(base, _BPW)], idx_v.at[pl.ds(f * _BPW, _BPW)], sem2
        )
        for f in range(_F)
    ]
    for cp in stage:
        cp.wait()

    copies = []
    for f in range(_F):
        sl = pl.ds(f * _BPW, _BPW)
        copies.append(pltpu.async_copy(tabs[f].at[idx_v.at[sl]], gath_v.at[sl], sem))
    for cp in copies:
        cp.wait()

    def _acc(c, carry):
        s = gath_v[pl.ds(c * _L, _L)]
        for f in range(1, _F):
            s = s + gath_v[pl.ds(f * _BPW + c * _L, _L)]
        out_v[pl.ds(c * _L, _L)] = s
        return carry

    lax.fori_loop(0, _CH, _acc, 0)

    pltpu.sync_copy(out_v, out_hbm.at[pl.ds(base, _BPW)])


def kernel(indices, tables):
    idx = indices.astype(jnp.int32)
    tabs = [tables[f, :, 0] for f in range(_F)]
    out = _lookup_sum(idx, *tabs)
    return out[:, None]


# speedup vs baseline: 1.6861x; 1.6861x over previous
import functools

import jax
import jax.numpy as jnp
from jax import lax
from jax.experimental import pallas as pl
from jax.experimental.pallas import tpu as pltpu
from jax.experimental.pallas import tpu_sc as plsc

_F = 26
_V1 = 100001
_VP = 100352
_B = 16384
_NC, _NS, _L = 2, 16, 16
_NW = _NC * _NS
_BPW = _B // _NW
_CH = _BPW // _L

_mesh = plsc.VectorSubcoreMesh(core_axis_name="c", subcore_axis_name="s")


@functools.partial(
    pl.kernel,
    out_type=jax.ShapeDtypeStruct((_B,), jnp.float32),
    mesh=_mesh,
    scratch_types=[
        pltpu.VMEM((_F * _BPW,), jnp.int32),
        pltpu.VMEM((_F * _BPW,), jnp.float32),
        pltpu.VMEM((_BPW,), jnp.float32),
        pltpu.SemaphoreType.DMA,
        pltpu.SemaphoreType.DMA,
    ],
    compiler_params=pltpu.CompilerParams(use_tc_tiling_on_sc=False),
)
def _lookup_sum(idx_hbm, tab_hbm, out_hbm, idx_v, gath_v, out_v, sem, sem2):
    wid = lax.axis_index("s") * _NC + lax.axis_index("c")
    base = wid * _BPW

    stage = [
        pltpu.async_copy(
            idx_hbm.at[f, pl.ds(base, _BPW)], idx_v.at[pl.ds(f * _BPW, _BPW)], sem2
        )
        for f in range(_F)
    ]
    for cp in stage:
        cp.wait()

    pltpu.async_copy(tab_hbm.at[idx_v], gath_v, sem).wait()

    def _acc(c, carry):
        s = gath_v[pl.ds(c * _L, _L)]
        for f in range(1, _F):
            s = s + gath_v[pl.ds(f * _BPW + c * _L, _L)]
        out_v[pl.ds(c * _L, _L)] = s
        return carry

    lax.fori_loop(0, _CH, _acc, 0)

    pltpu.sync_copy(out_v, out_hbm.at[pl.ds(base, _BPW)])


def kernel(indices, tables):
    idx = indices.astype(jnp.int32)
    offs = (jnp.arange(_F, dtype=jnp.int32) * _VP)[:, None]
    tab = jnp.pad(tables, ((0, 0), (0, _VP - _V1), (0, 0))).reshape(_F * _VP)
    out = _lookup_sum(idx + offs, tab)
    return out[:, None]


# speedup vs baseline: 1.7074x; 1.0126x over previous
import functools

import jax
import jax.numpy as jnp
from jax import lax
from jax.experimental import pallas as pl
from jax.experimental.pallas import tpu as pltpu
from jax.experimental.pallas import tpu_sc as plsc

_F = 26
_V1 = 100001
_VP = 100352
_B = 16384
_NC, _NS, _L = 2, 16, 16
_NW = _NC * _NS
_BPW = _B // _NW
_CH = _BPW // _L

_mesh = plsc.VectorSubcoreMesh(core_axis_name="c", subcore_axis_name="s")


@functools.partial(
    pl.kernel,
    out_type=jax.ShapeDtypeStruct((_B,), jnp.float32),
    mesh=_mesh,
    scratch_types=[
        pltpu.VMEM((_F * _BPW,), jnp.int32),
        pltpu.VMEM((_F * _BPW,), jnp.float32),
        pltpu.VMEM((_BPW,), jnp.float32),
        pltpu.SemaphoreType.DMA,
        pltpu.SemaphoreType.DMA,
    ],
    compiler_params=pltpu.CompilerParams(use_tc_tiling_on_sc=False),
)
def _lookup_sum(idx_hbm, tab_hbm, out_hbm, idx_v, gath_v, out_v, sem, sem2):
    wid = lax.axis_index("s") * _NC + lax.axis_index("c")
    base = wid * _BPW

    stage = [
        pltpu.async_copy(
            idx_hbm.at[pl.ds(f * _B + base, _BPW)],
            idx_v.at[pl.ds(f * _BPW, _BPW)],
            sem2,
        )
        for f in range(_F)
    ]
    copies = []
    for f in range(_F):
        stage[f].wait()
        sl = pl.ds(f * _BPW, _BPW)
        copies.append(pltpu.async_copy(tab_hbm.at[idx_v.at[sl]], gath_v.at[sl], sem))
    for cp in copies:
        cp.wait()

    def _acc(c, carry):
        s = gath_v[pl.ds(c * _L, _L)]
        for f in range(1, _F):
            s = s + gath_v[pl.ds(f * _BPW + c * _L, _L)]
        out_v[pl.ds(c * _L, _L)] = s
        return carry

    lax.fori_loop(0, _CH, _acc, 0)

    pltpu.sync_copy(out_v, out_hbm.at[pl.ds(base, _BPW)])


def kernel(indices, tables):
    idx = indices.astype(jnp.int32)
    offs = (jnp.arange(_F, dtype=jnp.int32) * _VP)[:, None]
    tab = jnp.pad(tables, ((0, 0), (0, _VP - _V1), (0, 0))).reshape(_F * _VP)
    out = _lookup_sum((idx + offs).reshape(_F * _B), tab)
    return out[:, None]
